# baseline (speedup 1.0000x reference)
import functools

import jax
import jax.numpy as jnp
from jax import lax
from jax.experimental import pallas as pl
from jax.experimental.pallas import tpu as pltpu
from jax.experimental.pallas import tpu_sc as plsc

EPS = 1e-5

_NC = 2
_NS = 16
_NW = _NC * _NS

_PW = 128
_K = 5
_RPG = 4
_SB = 512


@functools.cache
def _make_sc_repack(n, dim):
    rec_w = dim * _RPG
    nsuper = n // _SB
    tail_w = n - nsuper * _SB
    rows_s = _SB * dim // rec_w
    assert tail_w % 16 == 0

    mesh = plsc.VectorSubcoreMesh(core_axis_name="c", subcore_axis_name="s")

    @functools.partial(
        pl.kernel,
        mesh=mesh,
        compiler_params=pltpu.CompilerParams(needs_layout_passes=False),
        out_type=jax.ShapeDtypeStruct((n // _RPG, rec_w), jnp.float32),
        scratch_types=[
            pltpu.VMEM((2, dim, _SB), jnp.float32),
            pltpu.VMEM((2, rows_s, rec_w), jnp.float32),
            pltpu.VMEM((16, 17), jnp.float32),
            pltpu.SemaphoreType.DMA,
            pltpu.SemaphoreType.DMA,
        ],
    )
    def repack_kernel(tt_hbm, tail_hbm, out_hbm, in_v, out_v, skew_v,
                      sem_i, sem_o):
        wid = lax.axis_index("s") * _NC + lax.axis_index("c")
        trips = (nsuper - wid + _NW - 1) // _NW
        lanes = lax.iota(jnp.int32, 16)

        def col0_of(t):
            return pl.multiple_of((wid + t * _NW) * _SB, _SB)

        def fire_in(t, buf):
            pltpu.async_copy(tt_hbm.at[:, pl.ds(col0_of(t), _SB)],
                             in_v.at[buf], sem_i)

        @pl.when(trips > 0)
        def _():
            fire_in(0, 0)

        def sup(t, carry):
            buf = jnp.bitwise_and(t, 1)

            @pl.when(t + 1 < trips)
            def _():
                fire_in(t + 1, 1 - buf)

            pltpu.make_async_copy(tt_hbm.at[:, pl.ds(0, _SB)],
                                  in_v.at[buf], sem_i).wait()

            @pl.when(t >= 2)
            def _():
                pltpu.make_async_copy(out_v.at[buf],
                                      out_hbm.at[pl.ds(0, rows_s)],
                                      sem_o).wait()

            def blk(z, c2):
                cb = jnp.bitwise_and(z, (dim // 16) - 1) * 16
                ib = jnp.right_shift(z, 1) * 16
                for k in range(16):
                    skew_v[k, pl.ds(0, 16)] = in_v[buf, cb + k,
                                                   pl.ds(ib, 16)]
                for j in range(16):
                    g = plsc.load_gather(skew_v,
                                         [lanes, jnp.full((16,), j,
                                                          jnp.int32)])
                    r = ib + j
                    out_v[buf, jnp.right_shift(r, 2),
                          pl.ds(jnp.bitwise_and(r, _RPG - 1) * dim + cb,
                                16)] = g
                return c2

            lax.fori_loop(0, (dim // 16) * (_SB // 16), blk, 0)
            row0 = pl.multiple_of(col0_of(t) // _RPG, 8)
            pltpu.async_copy(out_v.at[buf],
                             out_hbm.at[pl.ds(row0, rows_s)], sem_o)
            return carry

        lax.fori_loop(0, trips, sup, 0)

        @pl.when(trips >= 2)
        def _():
            pltpu.make_async_copy(out_v.at[0],
                                  out_hbm.at[pl.ds(0, rows_s)],
                                  sem_o).wait()

        @pl.when(trips >= 1)
        def _():
            pltpu.make_async_copy(out_v.at[0],
                                  out_hbm.at[pl.ds(0, rows_s)],
                                  sem_o).wait()

        if tail_w:
            tr = tail_w // _RPG

            @pl.when(wid == 0)
            def _():
                pltpu.sync_copy(tail_hbm, out_v.at[0, pl.ds(0, tr)])
                pltpu.sync_copy(out_v.at[0, pl.ds(0, tr)],
                                out_hbm.at[pl.ds(nsuper * _SB // _RPG, tr)])

    return repack_kernel


@functools.cache
def _make_sc_gather(b, l, dim):
    rec_w = dim * _RPG
    assert b == _NW * _PW
    assert l % _K == 0
    nsteps = l // _K

    mesh = plsc.VectorSubcoreMesh(core_axis_name="c", subcore_axis_name="s")

    @functools.partial(
        pl.kernel,
        mesh=mesh,
        compiler_params=pltpu.CompilerParams(needs_layout_passes=False),
        out_type=jax.ShapeDtypeStruct((l, dim, b), jnp.float32),
        scratch_types=[
            pltpu.VMEM((l, _PW), jnp.int32),
            pltpu.VMEM((l, _PW), jnp.int32),
            pltpu.VMEM((_K * _PW, rec_w), jnp.float32),
            pltpu.VMEM((dim, _PW), jnp.float32),
            pltpu.VMEM((16, dim + 1), jnp.float32),
            pltpu.SemaphoreType.DMA,
        ],
    )
    def gather_kernel(idx_hbm, table_hbm, e_hbm, gidx_v, off_v, recs_v,
                      panel_v, skew_v, sem):
        wid = lax.axis_index("s") * _NC + lax.axis_index("c")
        b0 = wid * _PW
        pltpu.sync_copy(idx_hbm.at[:, pl.ds(b0, _PW)], gidx_v)

        def mk_idx(g, carry):
            for q in range(_PW // 16):
                v = gidx_v[g, pl.ds(q * 16, 16)]
                off_v[g, pl.ds(q * 16, 16)] = (
                    jnp.bitwise_and(v, _RPG - 1) * dim)
                gidx_v[g, pl.ds(q * 16, 16)] = jnp.right_shift(v, 2)
            return carry

        lax.fori_loop(0, l, mk_idx, 0)
        lanes = lax.iota(jnp.int32, 16)

        def step(s, carry):
            base_g = s * _K
            cps = []
            for j in range(_K):
                cps.append(pltpu.async_copy(
                    table_hbm.at[gidx_v.at[base_g + j]],
                    recs_v.at[pl.ds(j * _PW, _PW)],
                    sem))
            for j in range(_K):
                cps[j].wait()

                def grp(g, c2):
                    offs = off_v[base_g + j, pl.ds(g * 16, 16)]
                    row = j * _PW + g * 16
                    for k in range(16):
                        ofk = offs[k]
                        skew_v[k, pl.ds(0, 16)] = (
                            recs_v[row + k, pl.ds(ofk, 16)])
                        skew_v[k, pl.ds(16, 16)] = (
                            recs_v[row + k, pl.ds(ofk + 16, 16)])
                    for c in range(dim):
                        vals = plsc.load_gather(
                            skew_v, [lanes, jnp.full((16,), c, jnp.int32)])
                        panel_v[c, pl.ds(g * 16, 16)] = vals
                    return c2

                lax.fori_loop(0, _PW // 16, grp, 0)
                pltpu.sync_copy(panel_v,
                                e_hbm.at[base_g + j, :, pl.ds(b0, _PW)])
            return carry

        lax.fori_loop(0, nsteps, step, 0)

    return gather_kernel


@functools.cache
def _make_tc_dist(b, l, dim):
    nb = 512

    def body(e_ref, out_ref):
        e = e_ref[...]
        s = e[0:1]
        o = e[1:]
        sq = jnp.sum((o - s) ** 2, axis=1)
        un = jnp.sum(s * s, axis=1)
        vn = jnp.sum(o * o, axis=1)
        alpha = jnp.clip(1.0 - un, EPS, 1.0)
        beta = jnp.clip(1.0 - vn, EPS, 1.0)
        x = 1.0 + 2.0 * sq / (alpha * beta)
        x = jnp.maximum(x, 1.0 + EPS)
        out_ref[...] = jnp.log(x + jnp.sqrt((x - 1.0) * (x + 1.0)))

    return pl.pallas_call(
        body,
        grid=(b // nb,),
        in_specs=[pl.BlockSpec((l, dim, nb), lambda i: (0, 0, i))],
        out_specs=pl.BlockSpec((l - 1, nb), lambda i: (0, i)),
        out_shape=jax.ShapeDtypeStruct((l - 1, b), jnp.float32),
    )


def kernel(inputs, table):
    b, l = inputs.shape
    n, dim = table.shape
    idx_t = jnp.transpose(inputs).astype(jnp.int32)
    tt = jnp.transpose(table)
    n_main = (n // _SB) * _SB
    tail = table[n_main:].reshape((n - n_main) // _RPG, _RPG * dim)
    packed = _make_sc_repack(n, dim)(tt, tail)
    e_t = _make_sc_gather(b, l, dim)(idx_t, packed)
    dist_t = _make_tc_dist(b, l, dim)(e_t)
    e = jnp.transpose(e_t, (2, 0, 1))
    dist = jnp.transpose(dist_t)
    return dist, e


# speedup vs baseline: 1.7854x; 1.7854x over previous
import functools

import jax
import jax.numpy as jnp
from jax import lax
from jax.experimental import pallas as pl
from jax.experimental.pallas import tpu as pltpu
from jax.experimental.pallas import tpu_sc as plsc

EPS = 1e-5

_NC = 2
_NS = 16
_NW = _NC * _NS

_PW = 128
_K = 5
_RPG = 4


@functools.cache
def _make_sc_gather(b, l, dim):
    rec_w = dim * _RPG
    assert b == _NW * _PW
    assert l % _K == 0
    nsteps = l // _K

    mesh = plsc.VectorSubcoreMesh(core_axis_name="c", subcore_axis_name="s")

    @functools.partial(
        pl.kernel,
        mesh=mesh,
        compiler_params=pltpu.CompilerParams(needs_layout_passes=False),
        out_type=jax.ShapeDtypeStruct((l, dim, b), jnp.float32),
        scratch_types=[
            pltpu.VMEM((l, _PW), jnp.int32),
            pltpu.VMEM((l, _PW), jnp.int32),
            pltpu.VMEM((_K * _PW, rec_w), jnp.float32),
            pltpu.VMEM((2, dim, _PW), jnp.float32),
            pltpu.SemaphoreType.DMA,
            pltpu.SemaphoreType.DMA,
        ],
    )
    def gather_kernel(idx_hbm, table_hbm, e_hbm, gidx_v, off_v, recs_v,
                      panel_v, sem, sem_p):
        wid = lax.axis_index("s") * _NC + lax.axis_index("c")
        b0 = wid * _PW
        pltpu.sync_copy(idx_hbm.at[:, pl.ds(b0, _PW)], gidx_v)

        def mk_idx(g, carry):
            for q in range(_PW // 16):
                v = gidx_v[g, pl.ds(q * 16, 16)]
                off_v[g, pl.ds(q * 16, 16)] = (
                    jnp.bitwise_and(v, _RPG - 1) * dim)
                gidx_v[g, pl.ds(q * 16, 16)] = jnp.right_shift(v, 2)
            return carry

        lax.fori_loop(0, l, mk_idx, 0)
        lanes = lax.iota(jnp.int32, 16)

        def step(s, carry):
            base_g = s * _K
            cps = []
            for j in range(_K):
                cps.append(pltpu.async_copy(
                    table_hbm.at[gidx_v.at[base_g + j]],
                    recs_v.at[pl.ds(j * _PW, _PW)],
                    sem))
            for j in range(_K):
                cps[j].wait()
                jj = base_g + j
                slot = jnp.bitwise_and(jj, 1)

                @pl.when(jj >= 2)
                def _():
                    pltpu.make_async_copy(
                        panel_v.at[slot],
                        e_hbm.at[0, :, pl.ds(b0, _PW)], sem_p).wait()

                for g in range(_PW // 16):
                    rows = j * _PW + g * 16 + lanes
                    offs = off_v[jj, pl.ds(g * 16, 16)]
                    for c in range(dim):
                        vals = plsc.load_gather(recs_v, [rows, offs + c])
                        panel_v[slot, c, pl.ds(g * 16, 16)] = vals
                pltpu.async_copy(panel_v.at[slot],
                                 e_hbm.at[jj, :, pl.ds(b0, _PW)], sem_p)
            return carry

        lax.fori_loop(0, nsteps, step, 0)
        for _ in range(2):
            pltpu.make_async_copy(panel_v.at[0],
                                  e_hbm.at[0, :, pl.ds(b0, _PW)],
                                  sem_p).wait()

    return gather_kernel


@functools.cache
def _make_tc_dist(b, l, dim):
    nb = 512

    def body(e_ref, out_ref):
        e = e_ref[...]
        s = e[0:1]
        o = e[1:]
        sq = jnp.sum((o - s) ** 2, axis=1)
        un = jnp.sum(s * s, axis=1)
        vn = jnp.sum(o * o, axis=1)
        alpha = jnp.clip(1.0 - un, EPS, 1.0)
        beta = jnp.clip(1.0 - vn, EPS, 1.0)
        x = 1.0 + 2.0 * sq / (alpha * beta)
        x = jnp.maximum(x, 1.0 + EPS)
        out_ref[...] = jnp.log(x + jnp.sqrt((x - 1.0) * (x + 1.0)))

    return pl.pallas_call(
        body,
        grid=(b // nb,),
        in_specs=[pl.BlockSpec((l, dim, nb), lambda i: (0, 0, i))],
        out_specs=pl.BlockSpec((l - 1, nb), lambda i: (0, i)),
        out_shape=jax.ShapeDtypeStruct((l - 1, b), jnp.float32),
    )


def kernel(inputs, table):
    b, l = inputs.shape
    n, dim = table.shape
    idx_t = jnp.transpose(inputs).astype(jnp.int32)
    table2 = table.reshape(n // _RPG, dim * _RPG)
    e_t = _make_sc_gather(b, l, dim)(idx_t, table2)
    dist_t = _make_tc_dist(b, l, dim)(e_t)
    e = jnp.transpose(e_t, (2, 0, 1))
    dist = jnp.transpose(dist_t)
    return dist, e


# speedup vs baseline: 1.8259x; 1.0227x over previous
import functools

import jax
import jax.numpy as jnp
from jax import lax
from jax.experimental import pallas as pl
from jax.experimental.pallas import tpu as pltpu
from jax.experimental.pallas import tpu_sc as plsc

EPS = 1e-5

_NC = 2
_NS = 16
_NW = _NC * _NS

_PW = 128
_K = 5
_RPG = 4
_REC_MASK = 0
_REC_SHIFT = 0


@functools.cache
def _make_sc_gather(b, l, dim):
    rec_w = dim * _RPG
    assert b == _NW * _PW
    assert l % _K == 0
    nsteps = l // _K

    mesh = plsc.VectorSubcoreMesh(core_axis_name="c", subcore_axis_name="s")

    @functools.partial(
        pl.kernel,
        mesh=mesh,
        compiler_params=pltpu.CompilerParams(needs_layout_passes=False),
        out_type=jax.ShapeDtypeStruct((l, dim, b), jnp.float32),
        scratch_types=[
            pltpu.VMEM((l, _PW), jnp.int32),
            pltpu.VMEM((l, _PW), jnp.int32),
            pltpu.VMEM((_K * _PW, rec_w), jnp.float32),
            pltpu.VMEM((2, dim, _PW), jnp.float32),
            pltpu.SemaphoreType.DMA,
            pltpu.SemaphoreType.DMA,
        ],
    )
    def gather_kernel(idx_hbm, table_hbm, e_hbm, gidx_v, off_v, recs_v,
                      panel_v, sem, sem_p):
        wid = lax.axis_index("s") * _NC + lax.axis_index("c")
        b0 = wid * _PW
        pltpu.sync_copy(idx_hbm.at[:, pl.ds(b0, _PW)], gidx_v)

        def mk_idx(g, carry):
            for q in range(_PW // 16):
                v = gidx_v[g, pl.ds(q * 16, 16)]
                off_v[g, pl.ds(q * 16, 16)] = (
                    jnp.bitwise_and(v, _REC_MASK) * dim)
                gidx_v[g, pl.ds(q * 16, 16)] = jnp.right_shift(v, _REC_SHIFT)
            return carry

        lax.fori_loop(0, l, mk_idx, 0)
        lanes = lax.iota(jnp.int32, 16)

        def step(s, carry):
            base_g = s * _K
            cps = []
            for j in range(_K):
                cps.append(pltpu.async_copy(
                    table_hbm.at[gidx_v.at[base_g + j]],
                    recs_v.at[pl.ds(j * _PW, _PW)],
                    sem))
            for j in range(_K):
                cps[j].wait()
                jj = base_g + j
                slot = jnp.bitwise_and(jj, 1)

                @pl.when(jj >= 2)
                def _():
                    pltpu.make_async_copy(
                        panel_v.at[slot],
                        e_hbm.at[0, :, pl.ds(b0, _PW)], sem_p).wait()

                for g in range(_PW // 16):
                    rows = j * _PW + g * 16 + lanes
                    offs = off_v[jj, pl.ds(g * 16, 16)]
                    for c in range(dim):
                        vals = plsc.load_gather(recs_v, [rows, offs + c])
                        panel_v[slot, c, pl.ds(g * 16, 16)] = vals
                pltpu.async_copy(panel_v.at[slot],
                                 e_hbm.at[jj, :, pl.ds(b0, _PW)], sem_p)
            return carry

        lax.fori_loop(0, nsteps, step, 0)
        for _ in range(2):
            pltpu.make_async_copy(panel_v.at[0],
                                  e_hbm.at[0, :, pl.ds(b0, _PW)],
                                  sem_p).wait()

    return gather_kernel


@functools.cache
def _make_tc_dist(b, l, dim):
    nb = 512

    def body(e_ref, out_ref):
        e = e_ref[...]
        s = e[0:1]
        o = e[1:]
        sq = jnp.sum((o - s) ** 2, axis=1)
        un = jnp.sum(s * s, axis=1)
        vn = jnp.sum(o * o, axis=1)
        alpha = jnp.clip(1.0 - un, EPS, 1.0)
        beta = jnp.clip(1.0 - vn, EPS, 1.0)
        x = 1.0 + 2.0 * sq / (alpha * beta)
        x = jnp.maximum(x, 1.0 + EPS)
        out_ref[...] = jnp.log(x + jnp.sqrt((x - 1.0) * (x + 1.0)))

    return pl.pallas_call(
        body,
        grid=(b // nb,),
        in_specs=[pl.BlockSpec((l, dim, nb), lambda i: (0, 0, i))],
        out_specs=pl.BlockSpec((l - 1, nb), lambda i: (0, i)),
        out_shape=jax.ShapeDtypeStruct((l - 1, b), jnp.float32),
    )


def kernel(inputs, table):
    b, l = inputs.shape
    n, dim = table.shape
    idx_t = jnp.transpose(inputs).astype(jnp.int32)
    table2 = jnp.pad(table, ((0, 0), (0, _RPG * dim - dim)))
    e_t = _make_sc_gather(b, l, dim)(idx_t, table2)
    dist_t = _make_tc_dist(b, l, dim)(e_t)
    e = jnp.transpose(e_t, (2, 0, 1))
    dist = jnp.transpose(dist_t)
    return dist, e


# speedup vs baseline: 1.8274x; 1.0008x over previous
import functools

import jax
import jax.numpy as jnp
from jax import lax
from jax.experimental import pallas as pl
from jax.experimental.pallas import tpu as pltpu
from jax.experimental.pallas import tpu_sc as plsc

EPS = 1e-5

_NC = 2
_NS = 16
_NW = _NC * _NS

_PW = 128
_K = 5
_RPG = 4


@functools.cache
def _make_sc_gather(b, l, dim):
    rec_w = dim * _RPG
    assert b == _NW * _PW
    assert l % _K == 0
    nsteps = l // _K

    mesh = plsc.VectorSubcoreMesh(core_axis_name="c", subcore_axis_name="s")

    @functools.partial(
        pl.kernel,
        mesh=mesh,
        compiler_params=pltpu.CompilerParams(needs_layout_passes=False),
        out_type=jax.ShapeDtypeStruct((l, dim, b), jnp.float32),
        scratch_types=[
            pltpu.VMEM((l, _PW), jnp.int32),
            pltpu.VMEM((_K * _PW, rec_w), jnp.float32),
            pltpu.VMEM((2, dim, _PW), jnp.float32),
            pltpu.SemaphoreType.DMA,
            pltpu.SemaphoreType.DMA,
        ],
    )
    def gather_kernel(idx_hbm, table_hbm, e_hbm, gidx_v, recs_v,
                      panel_v, sem, sem_p):
        wid = lax.axis_index("s") * _NC + lax.axis_index("c")
        b0 = wid * _PW
        pltpu.sync_copy(idx_hbm.at[:, pl.ds(b0, _PW)], gidx_v)
        lanes = lax.iota(jnp.int32, 16)

        def step(s, carry):
            base_g = s * _K
            cps = []
            for j in range(_K):
                cps.append(pltpu.async_copy(
                    table_hbm.at[gidx_v.at[base_g + j]],
                    recs_v.at[pl.ds(j * _PW, _PW)],
                    sem))
            for j in range(_K):
                cps[j].wait()
                jj = base_g + j
                slot = jnp.bitwise_and(jj, 1)

                @pl.when(jj >= 2)
                def _():
                    pltpu.make_async_copy(
                        panel_v.at[slot],
                        e_hbm.at[0, :, pl.ds(b0, _PW)], sem_p).wait()

                for g in range(_PW // 16):
                    rows = j * _PW + g * 16 + lanes
                    for c in range(dim):
                        vals = plsc.load_gather(
                            recs_v, [rows, jnp.full((16,), c, jnp.int32)])
                        panel_v[slot, c, pl.ds(g * 16, 16)] = vals
                pltpu.async_copy(panel_v.at[slot],
                                 e_hbm.at[jj, :, pl.ds(b0, _PW)], sem_p)
            return carry

        lax.fori_loop(0, nsteps, step, 0)
        for _ in range(2):
            pltpu.make_async_copy(panel_v.at[0],
                                  e_hbm.at[0, :, pl.ds(b0, _PW)],
                                  sem_p).wait()

    return gather_kernel


@functools.cache
def _make_tc_dist(b, l, dim):
    nb = 512

    def body(e_ref, out_ref):
        e = e_ref[...]
        s = e[0:1]
        o = e[1:]
        sq = jnp.sum((o - s) ** 2, axis=1)
        un = jnp.sum(s * s, axis=1)
        vn = jnp.sum(o * o, axis=1)
        alpha = jnp.clip(1.0 - un, EPS, 1.0)
        beta = jnp.clip(1.0 - vn, EPS, 1.0)
        x = 1.0 + 2.0 * sq / (alpha * beta)
        x = jnp.maximum(x, 1.0 + EPS)
        out_ref[...] = jnp.log(x + jnp.sqrt((x - 1.0) * (x + 1.0)))

    return pl.pallas_call(
        body,
        grid=(b // nb,),
        in_specs=[pl.BlockSpec((l, dim, nb), lambda i: (0, 0, i))],
        out_specs=pl.BlockSpec((l - 1, nb), lambda i: (0, i)),
        out_shape=jax.ShapeDtypeStruct((l - 1, b), jnp.float32),
    )


def kernel(inputs, table):
    b, l = inputs.shape
    n, dim = table.shape
    idx_t = jnp.transpose(inputs).astype(jnp.int32)
    table2 = jnp.pad(table, ((0, 0), (0, _RPG * dim - dim)))
    e_t = _make_sc_gather(b, l, dim)(idx_t, table2)
    dist_t = _make_tc_dist(b, l, dim)(e_t)
    e = jnp.transpose(e_t, (2, 0, 1))
    dist = jnp.transpose(dist_t)
    return dist, e


# speedup vs baseline: 1.8791x; 1.0283x over previous
import functools

import jax
import jax.numpy as jnp
from jax import lax
from jax.experimental import pallas as pl
from jax.experimental.pallas import tpu as pltpu
from jax.experimental.pallas import tpu_sc as plsc

EPS = 1e-5

_NC = 2
_NS = 16
_NW = _NC * _NS

_PW = 128
_K = 5
_RPG = 4


@functools.cache
def _make_sc_gather(b, l, dim):
    rec_w = dim * _RPG
    assert b == _NW * _PW
    assert l % _K == 0
    nsteps = l // _K

    mesh = plsc.VectorSubcoreMesh(core_axis_name="c", subcore_axis_name="s")

    @functools.partial(
        pl.kernel,
        mesh=mesh,
        compiler_params=pltpu.CompilerParams(needs_layout_passes=False),
        out_type=jax.ShapeDtypeStruct((l, dim, b), jnp.float32),
        scratch_types=[
            pltpu.VMEM((l, _PW), jnp.int32),
            pltpu.VMEM((_K * _PW, rec_w), jnp.float32),
            pltpu.VMEM((2, dim, _PW), jnp.float32),
            pltpu.SemaphoreType.DMA,
            pltpu.SemaphoreType.DMA,
        ],
    )
    def gather_kernel(idx_hbm, table_hbm, e_hbm, gidx_v, recs_v,
                      panel_v, sem, sem_p):
        wid = lax.axis_index("s") * _NC + lax.axis_index("c")
        b0 = wid * _PW
        pltpu.sync_copy(idx_hbm.at[:, pl.ds(b0, _PW)], gidx_v)
        lanes = lax.iota(jnp.int32, 16)

        def fire(i, slot):
            pltpu.async_copy(table_hbm.at[gidx_v.at[i]],
                             recs_v.at[pl.ds(slot * _PW, _PW)], sem)

        for j in range(_K):
            fire(j, j)

        def step(i, carry):
            slot = i % _K
            pltpu.make_async_copy(
                table_hbm.at[gidx_v.at[0]],
                recs_v.at[pl.ds(slot * _PW, _PW)], sem).wait()
            pslot = jnp.bitwise_and(i, 1)

            @pl.when(i >= 2)
            def _():
                pltpu.make_async_copy(
                    panel_v.at[pslot],
                    e_hbm.at[0, :, pl.ds(b0, _PW)], sem_p).wait()

            base_r = slot * _PW
            for g in range(_PW // 16):
                rows = base_r + g * 16 + lanes
                for c in range(dim):
                    vals = plsc.load_gather(
                        recs_v, [rows, jnp.full((16,), c, jnp.int32)])
                    panel_v[pslot, c, pl.ds(g * 16, 16)] = vals
            pltpu.async_copy(panel_v.at[pslot],
                             e_hbm.at[i, :, pl.ds(b0, _PW)], sem_p)

            @pl.when(i + _K < l)
            def _():
                fire(i + _K, slot)

            return carry

        lax.fori_loop(0, l, step, 0)
        for _ in range(2):
            pltpu.make_async_copy(panel_v.at[0],
                                  e_hbm.at[0, :, pl.ds(b0, _PW)],
                                  sem_p).wait()

    return gather_kernel


@functools.cache
def _make_tc_dist(b, l, dim):
    nb = 512

    def body(e_ref, out_ref):
        e = e_ref[...]
        s = e[0:1]
        o = e[1:]
        sq = jnp.sum((o - s) ** 2, axis=1)
        un = jnp.sum(s * s, axis=1)
        vn = jnp.sum(o * o, axis=1)
        alpha = jnp.clip(1.0 - un, EPS, 1.0)
        beta = jnp.clip(1.0 - vn, EPS, 1.0)
        x = 1.0 + 2.0 * sq / (alpha * beta)
        x = jnp.maximum(x, 1.0 + EPS)
        out_ref[...] = jnp.log(x + jnp.sqrt((x - 1.0) * (x + 1.0)))

    return pl.pallas_call(
        body,
        grid=(b // nb,),
        in_specs=[pl.BlockSpec((l, dim, nb), lambda i: (0, 0, i))],
        out_specs=pl.BlockSpec((l - 1, nb), lambda i: (0, i)),
        out_shape=jax.ShapeDtypeStruct((l - 1, b), jnp.float32),
    )


def kernel(inputs, table):
    b, l = inputs.shape
    n, dim = table.shape
    idx_t = jnp.transpose(inputs).astype(jnp.int32)
    table2 = jnp.pad(table, ((0, 0), (0, _RPG * dim - dim)))
    e_t = _make_sc_gather(b, l, dim)(idx_t, table2)
    dist_t = _make_tc_dist(b, l, dim)(e_t)
    e = jnp.transpose(e_t, (2, 0, 1))
    dist = jnp.transpose(dist_t)
    return dist, e


# speedup vs baseline: 1.8799x; 1.0005x over previous
import functools

import jax
import jax.numpy as jnp
from jax import lax
from jax.experimental import pallas as pl
from jax.experimental.pallas import tpu as pltpu
from jax.experimental.pallas import tpu_sc as plsc

EPS = 1e-5

_NC = 2
_NS = 16
_NW = _NC * _NS

_PW = 128
_K = 5
_RPG = 4


@functools.cache
def _make_sc_gather(b, l, dim):
    rec_w = dim * _RPG
    assert b == _NW * _PW
    assert l % _K == 0

    mesh = plsc.VectorSubcoreMesh(core_axis_name="c", subcore_axis_name="s")

    @functools.partial(
        pl.kernel,
        mesh=mesh,
        compiler_params=pltpu.CompilerParams(needs_layout_passes=False),
        out_type=jax.ShapeDtypeStruct((l, dim, b), jnp.float32),
        scratch_types=[
            pltpu.VMEM((l, _PW), jnp.int32),
            pltpu.VMEM((_K * _PW, rec_w), jnp.float32),
            pltpu.VMEM((2, dim, _PW), jnp.float32),
            pltpu.SemaphoreType.DMA,
            pltpu.SemaphoreType.DMA,
        ],
    )
    def gather_kernel(idx_hbm, table_hbm, e_hbm, gidx_v, recs_v,
                      panel_v, sem, sem_p):
        wid = lax.axis_index("s") * _NC + lax.axis_index("c")
        b0 = wid * _PW
        pltpu.sync_copy(idx_hbm.at[:, pl.ds(b0, _PW)], gidx_v)
        lanes = lax.iota(jnp.int32, 16)

        def fire(i, slot):
            pltpu.async_copy(table_hbm.at[gidx_v.at[i]],
                             recs_v.at[pl.ds(slot * _PW, _PW)], sem)

        for j in range(_K):
            fire(j, j)

        def step(i, carry):
            slot = i % _K
            pltpu.make_async_copy(
                table_hbm.at[gidx_v.at[0]],
                recs_v.at[pl.ds(slot * _PW, _PW)], sem).wait()
            pslot = jnp.bitwise_and(i, 1)

            @pl.when(i >= 2)
            def _():
                pltpu.make_async_copy(
                    panel_v.at[pslot],
                    e_hbm.at[0, :, pl.ds(b0, _PW)], sem_p).wait()

            base_r = slot * _PW
            for g in range(_PW // 16):
                rows = base_r + g * 16 + lanes
                for c in range(dim):
                    vals = plsc.load_gather(
                        recs_v, [rows, jnp.full((16,), c, jnp.int32)])
                    panel_v[pslot, c, pl.ds(g * 16, 16)] = vals
            pltpu.async_copy(panel_v.at[pslot],
                             e_hbm.at[i, :, pl.ds(b0, _PW)], sem_p)

            @pl.when(i + _K < l)
            def _():
                fire(i + _K, slot)

            return carry

        lax.fori_loop(0, l, step, 0)
        for _ in range(2):
            pltpu.make_async_copy(panel_v.at[0],
                                  e_hbm.at[0, :, pl.ds(b0, _PW)],
                                  sem_p).wait()

    return gather_kernel


@functools.cache
def _make_tc_dist(b, l, dim):
    nb = 512

    def body(e_ref, out_ref):
        e = e_ref[...]
        s = e[0:1]
        o = e[1:]
        sq = jnp.sum((o - s) ** 2, axis=1)
        un = jnp.sum(s * s, axis=1)
        vn = jnp.sum(o * o, axis=1)
        alpha = jnp.clip(1.0 - un, EPS, 1.0)
        beta = jnp.clip(1.0 - vn, EPS, 1.0)
        x = 1.0 + 2.0 * sq / (alpha * beta)
        x = jnp.maximum(x, 1.0 + EPS)
        out_ref[...] = jnp.log(x + jnp.sqrt((x - 1.0) * (x + 1.0)))

    return pl.pallas_call(
        body,
        grid=(b // nb,),
        in_specs=[pl.BlockSpec((l, dim, nb), lambda i: (0, 0, i))],
        out_specs=pl.BlockSpec((l - 1, nb), lambda i: (0, i)),
        out_shape=jax.ShapeDtypeStruct((l - 1, b), jnp.float32),
    )


def kernel(inputs, table):
    b, l = inputs.shape
    n, dim = table.shape
    idx_t = jnp.transpose(inputs).astype(jnp.int32)
    table2 = jnp.pad(table, ((0, 0), (0, _RPG * dim - dim)))
    e_t = _make_sc_gather(b, l, dim)(idx_t, table2)
    dist_t = _make_tc_dist(b, l, dim)(e_t)
    e = jnp.transpose(e_t, (2, 0, 1))
    dist = jnp.transpose(dist_t)
    return dist, e
